# baseline (speedup 1.0000x reference)
import functools

import numpy as np

import jax
import jax.numpy as jnp
from jax import lax
from jax.experimental import pallas as pl
from jax.experimental.pallas import tpu as pltpu
from jax.experimental.pallas import tpu_sc as plsc

N = 8388608
NORM = 9.48933
NC = 2
NS = 16
L = 16
NW = NC * NS
CHUNK = 16384
NG = CHUNK // L

N_SC = 0
N_TC = N - N_SC
PER_W = max(N_SC // NW, CHUNK)
NCH = PER_W // CHUNK

BLKE = 2097152

_mesh = plsc.VectorSubcoreMesh(
    core_axis_name="c", subcore_axis_name="s", num_cores=NC, num_subcores=NS
)


@functools.partial(
    pl.kernel,
    out_type=jax.ShapeDtypeStruct((max(N_SC, 1),), jnp.float32),
    mesh=_mesh,
    scratch_types=[
        pltpu.VMEM((L,), jnp.float32),
        pltpu.VMEM((CHUNK,), jnp.int32),
        pltpu.VMEM((CHUNK,), jnp.int32),
        pltpu.VMEM((CHUNK,), jnp.int32),
        pltpu.VMEM((CHUNK,), jnp.int32),
        pltpu.VMEM((CHUNK,), jnp.int32),
        pltpu.VMEM((CHUNK,), jnp.float32),
        pltpu.VMEM((CHUNK,), jnp.float32),
        pltpu.SemaphoreType.DMA,
        pltpu.SemaphoreType.DMA,
        pltpu.SemaphoreType.DMA,
        pltpu.SemaphoreType.DMA,
        pltpu.SemaphoreType.DMA,
        pltpu.SemaphoreType.DMA,
        pltpu.SemaphoreType.DMA,
    ],
)
def _sc_lookup(an_hbm, table_hbm, out_hbm, table_v, ib0, ib1, ib2, ib3, ib4,
               ob0, ob1, si0, si1, si2, si3, si4, so0, so1):
    wid = lax.axis_index("c") * NS + lax.axis_index("s")
    base = wid * PER_W

    pltpu.sync_copy(table_hbm, table_v)
    tv = table_v[...] * NORM

    ibufs = (ib0, ib1, ib2, ib3, ib4)
    obufs = (ob0, ob1)
    isems = (si0, si1, si2, si3, si4)
    osems = (so0, so1)
    NBI = 5

    def start_in(c, slot):
        return pltpu.async_copy(
            an_hbm.at[pl.ds(N_TC + base + c * CHUNK, CHUNK)],
            ibufs[slot], isems[slot]
        )

    def start_out(c, slot):
        return pltpu.async_copy(
            obufs[slot], out_hbm.at[pl.ds(base + c * CHUNK, CHUNK)], osems[slot]
        )

    in_d = [None] * NBI
    out_d = [None, None]
    for p in range(min(NBI - 1, NCH)):
        in_d[p] = start_in(p, p)
    for c in range(NCH):
        cur = c % NBI
        nxt = c + NBI - 1
        if nxt < NCH:
            in_d[nxt % NBI] = start_in(nxt, nxt % NBI)
        in_d[cur].wait()
        if c >= 2:
            out_d[c & 1].wait()

        ib = ibufs[cur]
        ob = obufs[c & 1]

        @plsc.parallel_loop(0, NG, unroll=8)
        def _(g):
            idx = ib[pl.ds(g * L, L)]
            ob[pl.ds(g * L, L)] = tv.at[idx].get(mode="promise_in_bounds")

        out_d[c & 1] = start_out(c, c & 1)

    out_d[(NCH - 1) & 1].wait()
    if NCH > 1:
        out_d[NCH & 1].wait()


def _tc_body(tab_ref, an_ref, out_ref):
    a = an_ref[...]
    acc = jnp.where(a == 0, tab_ref[0] * NORM, 0.0)
    for i in range(1, 10):
        acc = jnp.where(a == i, tab_ref[i] * NORM, acc)
    out_ref[...] = acc


_tc_lookup = pl.pallas_call(
    _tc_body,
    grid=(N_TC // BLKE if N_TC else 1,),
    in_specs=[
        pl.BlockSpec(memory_space=pltpu.SMEM),
        pl.BlockSpec((BLKE,), lambda i: (i,)),
    ],
    out_specs=pl.BlockSpec((BLKE,), lambda i: (i,)),
    out_shape=jax.ShapeDtypeStruct((N,), jnp.float32),
)

_VINV_EVEN = np.linalg.inv(np.vander(np.arange(0, 10, 2), increasing=True))
_VINV_ODD = np.linalg.inv(np.vander(np.arange(1, 10, 2), increasing=True))

TCH = 524288
TNCH = N // TCH
TNB = 4


def _tc_body2(tab_ref, an_hbm, out_hbm, ib0, ib1, ib2, ib3, ob0, ob1, ob2,
              ob3, isems, osems):
    ce = [tab_ref[i] for i in range(5)]
    co = [tab_ref[5 + i] for i in range(5)]
    ibs = (ib0, ib1, ib2, ib3)
    obs = (ob0, ob1, ob2, ob3)

    def start_in(c):
        return pltpu.make_async_copy(
            an_hbm.at[pl.ds(c * TCH, TCH)], ibs[c % TNB], isems.at[c % TNB]
        )

    def start_out(c):
        return pltpu.make_async_copy(
            obs[c % TNB], out_hbm.at[pl.ds(c * TCH, TCH)], osems.at[c % TNB]
        )

    for p in range(TNB - 1):
        start_in(p).start()
    for c in range(TNCH):
        nxt = c + TNB - 1
        if nxt < TNCH:
            start_in(nxt).start()
        start_in(c).wait()
        if c >= TNB:
            start_out(c - TNB).wait()
        a = ibs[c % TNB][...]
        af = a.astype(jnp.float32)
        pe = jnp.full(a.shape, ce[4], dtype=jnp.float32)
        po = jnp.full(a.shape, co[4], dtype=jnp.float32)
        for k in (3, 2, 1, 0):
            pe = pe * af + ce[k]
            po = po * af + co[k]
        obs[c % TNB][...] = jnp.where((a & 1) == 0, pe, po)
        start_out(c).start()
    for c in range(max(TNCH - TNB, 0), TNCH):
        start_out(c).wait()


_tc_lookup2 = pl.pallas_call(
    _tc_body2,
    in_specs=[
        pl.BlockSpec(memory_space=pltpu.SMEM),
        pl.BlockSpec(memory_space=pl.ANY),
    ],
    out_specs=pl.BlockSpec(memory_space=pl.ANY),
    out_shape=jax.ShapeDtypeStruct((N,), jnp.float32),
    scratch_shapes=[
        pltpu.VMEM((TCH,), jnp.int32),
        pltpu.VMEM((TCH,), jnp.int32),
        pltpu.VMEM((TCH,), jnp.int32),
        pltpu.VMEM((TCH,), jnp.int32),
        pltpu.VMEM((TCH,), jnp.float32),
        pltpu.VMEM((TCH,), jnp.float32),
        pltpu.VMEM((TCH,), jnp.float32),
        pltpu.VMEM((TCH,), jnp.float32),
        pltpu.SemaphoreType.DMA((TNB,)),
        pltpu.SemaphoreType.DMA((TNB,)),
    ],
)


def kernel(atomic_numbers, element_charges):
    t = element_charges.astype(jnp.float32) * NORM
    ce = jnp.asarray(_VINV_EVEN, jnp.float32) @ t[0::2]
    co = jnp.asarray(_VINV_ODD, jnp.float32) @ t[1::2]
    table16 = jnp.concatenate([ce, co, jnp.zeros(6, jnp.float32)])
    full = _tc_lookup2(table16, atomic_numbers)
    if N_SC == 0:
        return full
    sc_part = _sc_lookup(atomic_numbers, table16)
    return lax.dynamic_update_slice(full, sc_part, (N_TC,))


# speedup vs baseline: 1.1903x; 1.1903x over previous
import functools

import jax
import jax.numpy as jnp
from jax import lax
from jax.experimental import pallas as pl
from jax.experimental.pallas import tpu as pltpu
from jax.experimental.pallas import tpu_sc as plsc

N = 8388608
NORM = 9.48933
NC = 2
NS = 16
L = 16
NW = NC * NS
CHUNK = 16384
NG = CHUNK // L

N_SC = 0
N_TC = N - N_SC
PER_W = max(N_SC // NW, CHUNK)
NCH = PER_W // CHUNK

BLKE = 2097152

_mesh = plsc.VectorSubcoreMesh(
    core_axis_name="c", subcore_axis_name="s", num_cores=NC, num_subcores=NS
)


@functools.partial(
    pl.kernel,
    out_type=jax.ShapeDtypeStruct((max(N_SC, 1),), jnp.float32),
    mesh=_mesh,
    scratch_types=[
        pltpu.VMEM((L,), jnp.float32),
        pltpu.VMEM((CHUNK,), jnp.int32),
        pltpu.VMEM((CHUNK,), jnp.int32),
        pltpu.VMEM((CHUNK,), jnp.int32),
        pltpu.VMEM((CHUNK,), jnp.int32),
        pltpu.VMEM((CHUNK,), jnp.int32),
        pltpu.VMEM((CHUNK,), jnp.float32),
        pltpu.VMEM((CHUNK,), jnp.float32),
        pltpu.SemaphoreType.DMA,
        pltpu.SemaphoreType.DMA,
        pltpu.SemaphoreType.DMA,
        pltpu.SemaphoreType.DMA,
        pltpu.SemaphoreType.DMA,
        pltpu.SemaphoreType.DMA,
        pltpu.SemaphoreType.DMA,
    ],
)
def _sc_lookup(an_hbm, table_hbm, out_hbm, table_v, ib0, ib1, ib2, ib3, ib4,
               ob0, ob1, si0, si1, si2, si3, si4, so0, so1):
    wid = lax.axis_index("c") * NS + lax.axis_index("s")
    base = wid * PER_W

    pltpu.sync_copy(table_hbm, table_v)
    tv = table_v[...] * NORM

    ibufs = (ib0, ib1, ib2, ib3, ib4)
    obufs = (ob0, ob1)
    isems = (si0, si1, si2, si3, si4)
    osems = (so0, so1)
    NBI = 5

    def start_in(c, slot):
        return pltpu.async_copy(
            an_hbm.at[pl.ds(N_TC + base + c * CHUNK, CHUNK)],
            ibufs[slot], isems[slot]
        )

    def start_out(c, slot):
        return pltpu.async_copy(
            obufs[slot], out_hbm.at[pl.ds(base + c * CHUNK, CHUNK)], osems[slot]
        )

    in_d = [None] * NBI
    out_d = [None, None]
    for p in range(min(NBI - 1, NCH)):
        in_d[p] = start_in(p, p)
    for c in range(NCH):
        cur = c % NBI
        nxt = c + NBI - 1
        if nxt < NCH:
            in_d[nxt % NBI] = start_in(nxt, nxt % NBI)
        in_d[cur].wait()
        if c >= 2:
            out_d[c & 1].wait()

        ib = ibufs[cur]
        ob = obufs[c & 1]

        @plsc.parallel_loop(0, NG, unroll=8)
        def _(g):
            idx = ib[pl.ds(g * L, L)]
            ob[pl.ds(g * L, L)] = tv.at[idx].get(mode="promise_in_bounds")

        out_d[c & 1] = start_out(c, c & 1)

    out_d[(NCH - 1) & 1].wait()
    if NCH > 1:
        out_d[NCH & 1].wait()


def _tc_body(tab_ref, an_ref, out_ref):
    a = an_ref[...]
    acc = jnp.where(a == 0, tab_ref[0] * NORM, 0.0)
    for i in range(1, 10):
        acc = jnp.where(a == i, tab_ref[i] * NORM, acc)
    out_ref[...] = acc


_tc_lookup = pl.pallas_call(
    _tc_body,
    grid=(N_TC // BLKE if N_TC else 1,),
    in_specs=[
        pl.BlockSpec(memory_space=pltpu.SMEM),
        pl.BlockSpec((BLKE,), lambda i: (i,)),
    ],
    out_specs=pl.BlockSpec((BLKE,), lambda i: (i,)),
    out_shape=jax.ShapeDtypeStruct((N,), jnp.float32),
)

TCH = 524288
TNCH = N // TCH
TNB = 4


def _tc_body2(tab_ref, an_hbm, out_hbm, ib0, ib1, ib2, ib3, ob0, ob1, ob2,
              ob3, isems, osems):
    t = [tab_ref[i] * NORM for i in range(10)]
    ibs = (ib0, ib1, ib2, ib3)
    obs = (ob0, ob1, ob2, ob3)

    def start_in(c):
        return pltpu.make_async_copy(
            an_hbm.at[pl.ds(c * TCH, TCH)], ibs[c % TNB], isems.at[c % TNB]
        )

    def start_out(c):
        return pltpu.make_async_copy(
            obs[c % TNB], out_hbm.at[pl.ds(c * TCH, TCH)], osems.at[c % TNB]
        )

    for p in range(TNB - 1):
        start_in(p).start()
    for c in range(TNCH):
        nxt = c + TNB - 1
        if nxt < TNCH:
            start_in(nxt).start()
        start_in(c).wait()
        if c >= TNB:
            start_out(c - TNB).wait()
        a = ibs[c % TNB][...]
        h = a >> 1
        u = jnp.full(a.shape, t[0], dtype=jnp.float32)
        v = jnp.full(a.shape, t[1], dtype=jnp.float32)
        for j in range(1, 5):
            m = h == j
            u = jnp.where(m, t[2 * j], u)
            v = jnp.where(m, t[2 * j + 1], v)
        obs[c % TNB][...] = jnp.where((a & 1) == 0, u, v)
        start_out(c).start()
    for c in range(max(TNCH - TNB, 0), TNCH):
        start_out(c).wait()


_tc_lookup2 = pl.pallas_call(
    _tc_body2,
    in_specs=[
        pl.BlockSpec(memory_space=pltpu.SMEM),
        pl.BlockSpec(memory_space=pl.ANY),
    ],
    out_specs=pl.BlockSpec(memory_space=pl.ANY),
    out_shape=jax.ShapeDtypeStruct((N,), jnp.float32),
    scratch_shapes=[
        pltpu.VMEM((TCH,), jnp.int32),
        pltpu.VMEM((TCH,), jnp.int32),
        pltpu.VMEM((TCH,), jnp.int32),
        pltpu.VMEM((TCH,), jnp.int32),
        pltpu.VMEM((TCH,), jnp.float32),
        pltpu.VMEM((TCH,), jnp.float32),
        pltpu.VMEM((TCH,), jnp.float32),
        pltpu.VMEM((TCH,), jnp.float32),
        pltpu.SemaphoreType.DMA((TNB,)),
        pltpu.SemaphoreType.DMA((TNB,)),
    ],
)


def kernel(atomic_numbers, element_charges):
    table16 = jnp.pad(element_charges.astype(jnp.float32), (0, L - 10))
    full = _tc_lookup2(table16, atomic_numbers)
    if N_SC == 0:
        return full
    sc_part = _sc_lookup(atomic_numbers, table16)
    return lax.dynamic_update_slice(full, sc_part, (N_TC,))


# speedup vs baseline: 1.2028x; 1.0106x over previous
import functools

import jax
import jax.numpy as jnp
from jax import lax
from jax.experimental import pallas as pl
from jax.experimental.pallas import tpu as pltpu
from jax.experimental.pallas import tpu_sc as plsc

N = 8388608
NORM = 9.48933
NC = 2
NS = 16
L = 16
NW = NC * NS
CHUNK = 16384
NG = CHUNK // L

N_SC = 0
N_TC = N - N_SC
PER_W = max(N_SC // NW, CHUNK)
NCH = PER_W // CHUNK

BLKE = 2097152

_mesh = plsc.VectorSubcoreMesh(
    core_axis_name="c", subcore_axis_name="s", num_cores=NC, num_subcores=NS
)


@functools.partial(
    pl.kernel,
    out_type=jax.ShapeDtypeStruct((max(N_SC, 1),), jnp.float32),
    mesh=_mesh,
    scratch_types=[
        pltpu.VMEM((L,), jnp.float32),
        pltpu.VMEM((CHUNK,), jnp.int32),
        pltpu.VMEM((CHUNK,), jnp.int32),
        pltpu.VMEM((CHUNK,), jnp.int32),
        pltpu.VMEM((CHUNK,), jnp.int32),
        pltpu.VMEM((CHUNK,), jnp.int32),
        pltpu.VMEM((CHUNK,), jnp.float32),
        pltpu.VMEM((CHUNK,), jnp.float32),
        pltpu.SemaphoreType.DMA,
        pltpu.SemaphoreType.DMA,
        pltpu.SemaphoreType.DMA,
        pltpu.SemaphoreType.DMA,
        pltpu.SemaphoreType.DMA,
        pltpu.SemaphoreType.DMA,
        pltpu.SemaphoreType.DMA,
    ],
)
def _sc_lookup(an_hbm, table_hbm, out_hbm, table_v, ib0, ib1, ib2, ib3, ib4,
               ob0, ob1, si0, si1, si2, si3, si4, so0, so1):
    wid = lax.axis_index("c") * NS + lax.axis_index("s")
    base = wid * PER_W

    pltpu.sync_copy(table_hbm, table_v)
    tv = table_v[...] * NORM

    ibufs = (ib0, ib1, ib2, ib3, ib4)
    obufs = (ob0, ob1)
    isems = (si0, si1, si2, si3, si4)
    osems = (so0, so1)
    NBI = 5

    def start_in(c, slot):
        return pltpu.async_copy(
            an_hbm.at[pl.ds(N_TC + base + c * CHUNK, CHUNK)],
            ibufs[slot], isems[slot]
        )

    def start_out(c, slot):
        return pltpu.async_copy(
            obufs[slot], out_hbm.at[pl.ds(base + c * CHUNK, CHUNK)], osems[slot]
        )

    in_d = [None] * NBI
    out_d = [None, None]
    for p in range(min(NBI - 1, NCH)):
        in_d[p] = start_in(p, p)
    for c in range(NCH):
        cur = c % NBI
        nxt = c + NBI - 1
        if nxt < NCH:
            in_d[nxt % NBI] = start_in(nxt, nxt % NBI)
        in_d[cur].wait()
        if c >= 2:
            out_d[c & 1].wait()

        ib = ibufs[cur]
        ob = obufs[c & 1]

        @plsc.parallel_loop(0, NG, unroll=8)
        def _(g):
            idx = ib[pl.ds(g * L, L)]
            ob[pl.ds(g * L, L)] = tv.at[idx].get(mode="promise_in_bounds")

        out_d[c & 1] = start_out(c, c & 1)

    out_d[(NCH - 1) & 1].wait()
    if NCH > 1:
        out_d[NCH & 1].wait()


def _tc_body(tab_ref, an_ref, out_ref):
    a = an_ref[...]
    acc = jnp.where(a == 0, tab_ref[0] * NORM, 0.0)
    for i in range(1, 10):
        acc = jnp.where(a == i, tab_ref[i] * NORM, acc)
    out_ref[...] = acc


_tc_lookup = pl.pallas_call(
    _tc_body,
    grid=(N_TC // BLKE if N_TC else 1,),
    in_specs=[
        pl.BlockSpec(memory_space=pltpu.SMEM),
        pl.BlockSpec((BLKE,), lambda i: (i,)),
    ],
    out_specs=pl.BlockSpec((BLKE,), lambda i: (i,)),
    out_shape=jax.ShapeDtypeStruct((N,), jnp.float32),
)

TCH = 1048576
TNCH = N // TCH
TNB = 4


def _tc_body2(tab_ref, an_hbm, out_hbm, ib0, ib1, ib2, ib3, ob0, ob1, ob2,
              ob3, isems, osems):
    t = [tab_ref[i] * NORM for i in range(10)]
    ibs = (ib0, ib1, ib2, ib3)
    obs = (ob0, ob1, ob2, ob3)

    def start_in(c):
        return pltpu.make_async_copy(
            an_hbm.at[pl.ds(c * TCH, TCH)], ibs[c % TNB], isems.at[c % TNB]
        )

    def start_out(c):
        return pltpu.make_async_copy(
            obs[c % TNB], out_hbm.at[pl.ds(c * TCH, TCH)], osems.at[c % TNB]
        )

    for p in range(TNB - 1):
        start_in(p).start()
    for c in range(TNCH):
        nxt = c + TNB - 1
        if nxt < TNCH:
            start_in(nxt).start()
        start_in(c).wait()
        if c >= TNB:
            start_out(c - TNB).wait()
        a = ibs[c % TNB][...]
        h = a >> 1
        u = jnp.full(a.shape, t[0], dtype=jnp.float32)
        v = jnp.full(a.shape, t[1], dtype=jnp.float32)
        for j in range(1, 5):
            m = h == j
            u = jnp.where(m, t[2 * j], u)
            v = jnp.where(m, t[2 * j + 1], v)
        obs[c % TNB][...] = jnp.where((a & 1) == 0, u, v)
        start_out(c).start()
    for c in range(max(TNCH - TNB, 0), TNCH):
        start_out(c).wait()


_tc_lookup2 = pl.pallas_call(
    _tc_body2,
    in_specs=[
        pl.BlockSpec(memory_space=pltpu.SMEM),
        pl.BlockSpec(memory_space=pl.ANY),
    ],
    out_specs=pl.BlockSpec(memory_space=pl.ANY),
    out_shape=jax.ShapeDtypeStruct((N,), jnp.float32),
    scratch_shapes=[
        pltpu.VMEM((TCH,), jnp.int32),
        pltpu.VMEM((TCH,), jnp.int32),
        pltpu.VMEM((TCH,), jnp.int32),
        pltpu.VMEM((TCH,), jnp.int32),
        pltpu.VMEM((TCH,), jnp.float32),
        pltpu.VMEM((TCH,), jnp.float32),
        pltpu.VMEM((TCH,), jnp.float32),
        pltpu.VMEM((TCH,), jnp.float32),
        pltpu.SemaphoreType.DMA((TNB,)),
        pltpu.SemaphoreType.DMA((TNB,)),
    ],
)


def kernel(atomic_numbers, element_charges):
    table16 = jnp.pad(element_charges.astype(jnp.float32), (0, L - 10))
    full = _tc_lookup2(table16, atomic_numbers)
    if N_SC == 0:
        return full
    sc_part = _sc_lookup(atomic_numbers, table16)
    return lax.dynamic_update_slice(full, sc_part, (N_TC,))
